# baseline (speedup 1.0000x reference)
import jax
import jax.numpy as jnp
from jax.experimental import pallas as pl
from jax.experimental.pallas import tpu as pltpu

M, N, B = 512, 2047, 64
NP = 2048
NUM_LAYERS = 16
RHO = 0.5
MAX_DEPTH = 10


def _matmul_kernel(y_ref, at_ref, apt_ref, w_ref, sc_ref, x_ref, xp_ref,
                   apr_ref, u_ref):
    y = y_ref[...]
    A_T = at_ref[...]
    x = x_ref[...]
    x_prev = xp_ref[...]
    sigc2 = sc_ref[0, 1]

    residual = y - jnp.dot(x, A_T, preferred_element_type=jnp.float32)
    apr_ref[...] = jnp.dot(residual, apt_ref[...],
                           preferred_element_type=jnp.float32)
    active = jnp.sum((jnp.abs(x) > 1e-6).astype(jnp.float32),
                     axis=-1, keepdims=True)
    beta = sigc2 * (active / float(N))
    z = x + beta * (x - x_prev)
    residual2 = y - jnp.dot(z, A_T, preferred_element_type=jnp.float32)
    u_ref[...] = z + jnp.dot(residual2, w_ref[...],
                             preferred_element_type=jnp.float32)


def _select_kernel(u_ref, dw_ref, anc_ref, ts_ref, kv_ref, out_ref):
    u = u_ref[...]
    dw = dw_ref[...]
    theta_s = ts_ref[0, 0]
    Kv = kv_ref[0, 0].astype(jnp.int32)

    lane = jax.lax.broadcasted_iota(jnp.int32, (1, NP), 1)
    valid = lane < N

    s = jnp.abs(u) * dw
    s_bits = jax.lax.bitcast_convert_type(s, jnp.int32)
    s_bits = jnp.where(valid, s_bits, -1)

    def cnt_ge(t):
        return jnp.sum((s_bits >= t).astype(jnp.int32),
                       axis=-1, keepdims=True)

    p = jnp.where(cnt_ge(jnp.full((B, 1), 1 << 30, jnp.int32)) >= Kv,
                  jnp.full((B, 1), 1 << 30, jnp.int32),
                  jnp.zeros((B, 1), jnp.int32))
    for shift in range(27, -1, -3):
        d = (cnt_ge(p + (1 << shift)) >= Kv).astype(jnp.int32)
        for k in range(2, 8):
            d = d + (cnt_ge(p + (k << shift)) >= Kv).astype(jnp.int32)
        p = p + (d << shift)
    v_bits = p
    gt = s_bits > v_bits
    eq = s_bits == v_bits
    c_gt = jnp.sum(gt.astype(jnp.int32), axis=-1, keepdims=True)
    need = Kv - c_gt

    def cnt_eq_le(t):
        return jnp.sum((eq & (lane <= t)).astype(jnp.int32),
                       axis=-1, keepdims=True)

    q = jnp.zeros((B, 1), jnp.int32)
    for shift in range(9, -1, -3):
        low1 = (1 << shift) - 1
        d = (cnt_eq_le(q + low1) < need).astype(jnp.int32)
        for k in range(1, 7):
            d = d + (cnt_eq_le(q + (k << shift) + low1) <
                     need).astype(jnp.int32)
        q = q + (d << shift)
    mask = (gt | (eq & (lane <= q))).astype(jnp.bfloat16)

    closed = jnp.dot(mask, anc_ref[...], preferred_element_type=jnp.float32)
    maskf = (closed > 0.5).astype(jnp.float32)

    x_new = jnp.sign(u) * jnp.maximum(jnp.abs(u) - theta_s, 0.0) * maskf
    out_ref[...] = jnp.where(valid, x_new, 0.0)


def _pad_cols(a, np_):
    return jnp.pad(a, ((0, 0), (0, np_ - a.shape[1])))


def kernel(y, A, W, A_pinv, c1, c2, c3, parent, depth):
    A_T = jnp.pad(A.T, ((0, NP - N), (0, 0)))
    Apinv_T = _pad_cols(A_pinv.T, NP)
    Wp = _pad_cols(W, NP)
    dw = _pad_cols((RHO ** depth.astype(jnp.float32))[None, :], NP)

    cur = jnp.arange(N, dtype=jnp.int32)
    aa = jnp.arange(N, dtype=jnp.int32)[None, :]
    anc = jnp.zeros((N, N), jnp.bool_)
    for _ in range(MAX_DEPTH + 1):
        anc = anc | (cur[:, None] == aa)
        cur = parent[cur]
    anc_bf = jnp.pad(anc.astype(jnp.bfloat16),
                     ((0, NP - N), (0, NP - N)))

    c1a = jnp.abs(c1)
    c3a = jnp.abs(c3)
    sc = jnp.stack([c1a[0], jax.nn.sigmoid(c2[0]),
                    c3a[0], jnp.float32(0.0)]).reshape(1, 4)

    matmul_call = pl.pallas_call(
        _matmul_kernel,
        out_shape=(jax.ShapeDtypeStruct((B, NP), jnp.float32),
                   jax.ShapeDtypeStruct((B, NP), jnp.float32)),
    )
    select_call = pl.pallas_call(
        _select_kernel,
        out_shape=jax.ShapeDtypeStruct((B, NP), jnp.float32),
    )

    Apinv_y = (A_pinv @ y.T).T
    y_l1 = jnp.clip(jnp.sum(jnp.abs(Apinv_y), axis=-1, keepdims=True),
                    1e-12, None)

    x = jnp.zeros((B, NP), jnp.float32)
    x_prev = jnp.zeros_like(x)
    for k in range(NUM_LAYERS):
        Apinv_res, u = matmul_call(y, A_T, Apinv_T, Wp, sc, x, x_prev)
        Apr = Apinv_res[:, :N]
        res_l1 = jnp.clip(jnp.sum(jnp.abs(Apr), axis=-1, keepdims=True),
                          1e-12, None)
        residual_ratio = jnp.clip(res_l1 / y_l1, 0.0, 1.0)
        theta = c1a * residual_ratio
        layer_progress = float(k + 1) / float(NUM_LAYERS)
        ratio = jnp.clip(y_l1 / res_l1, 1.0, None)
        log_ratio = jnp.clip(jnp.log(ratio), 0.0, None)
        signal_estimate = jax.nn.sigmoid(log_ratio - 1.0)
        K_tree = jnp.clip(c3a * float(N) *
                          jnp.maximum(signal_estimate, layer_progress),
                          1.0, float(N) * 0.6)
        theta_s = jnp.mean(theta)
        Kv = jnp.maximum(jnp.floor(jnp.mean(K_tree)), 1.0)
        x_new = select_call(u, dw, anc_bf, theta_s.reshape(1, 1),
                            Kv.reshape(1, 1))
        x_prev = x
        x = x_new
    return x[:, :N]


# speedup vs baseline: 1.5308x; 1.5308x over previous
import jax
import jax.numpy as jnp
from jax.experimental import pallas as pl
from jax.experimental.pallas import tpu as pltpu

M, N, B = 512, 2047, 64
NP = 2048
NUM_LAYERS = 16
RHO = 0.5
MAX_DEPTH = 10


def _matmul_kernel(y_ref, at_ref, apt_ref, w_ref, sc_ref, x_ref, xp_ref,
                   apr_ref, u_ref):
    y = y_ref[...]
    A_T = at_ref[...]
    x = x_ref[...]
    x_prev = xp_ref[...]
    sigc2 = sc_ref[0, 1]

    residual = y - jnp.dot(x, A_T, preferred_element_type=jnp.float32)
    apr_ref[...] = jnp.dot(residual, apt_ref[...],
                           preferred_element_type=jnp.float32)
    active = jnp.sum((jnp.abs(x) > 1e-6).astype(jnp.float32),
                     axis=-1, keepdims=True)
    beta = sigc2 * (active / float(N))
    z = x + beta * (x - x_prev)
    residual2 = y - jnp.dot(z, A_T, preferred_element_type=jnp.float32)
    u_ref[...] = z + jnp.dot(residual2, w_ref[...],
                             preferred_element_type=jnp.float32)


def _select_kernel(u_ref, dw_ref, s10_ref, s9_ref, s8_ref, anc_ref, ts_ref,
                   kv_ref, out_ref):
    u = u_ref[...]
    dw = dw_ref[...]
    theta_s = ts_ref[0, 0]
    Kv = kv_ref[0, 0].astype(jnp.int32)

    lane = jax.lax.broadcasted_iota(jnp.int32, (1, NP), 1)
    valid = lane < N

    s = jnp.abs(u) * dw
    s_bits = jax.lax.bitcast_convert_type(s, jnp.int32)
    s_bits = jnp.where(valid, s_bits, -1)

    def cnt_ge(t):
        return jnp.sum((s_bits >= t).astype(jnp.int32),
                       axis=-1, keepdims=True)

    p = jnp.where(cnt_ge(jnp.full((B, 1), 1 << 30, jnp.int32)) >= Kv,
                  jnp.full((B, 1), 1 << 30, jnp.int32),
                  jnp.zeros((B, 1), jnp.int32))
    for shift in range(27, -1, -3):
        d = (cnt_ge(p + (1 << shift)) >= Kv).astype(jnp.int32)
        for k in range(2, 8):
            d = d + (cnt_ge(p + (k << shift)) >= Kv).astype(jnp.int32)
        p = p + (d << shift)
    v_bits = p
    gt = s_bits > v_bits
    eq = s_bits == v_bits
    c_gt = jnp.sum(gt.astype(jnp.int32), axis=-1, keepdims=True)
    need = Kv - c_gt

    def cnt_eq_le(t):
        return jnp.sum((eq & (lane <= t)).astype(jnp.int32),
                       axis=-1, keepdims=True)

    q = jnp.zeros((B, 1), jnp.int32)
    for shift in range(9, -1, -3):
        low1 = (1 << shift) - 1
        d = (cnt_eq_le(q + low1) < need).astype(jnp.int32)
        for k in range(1, 7):
            d = d + (cnt_eq_le(q + (k << shift) + low1) <
                     need).astype(jnp.int32)
        q = q + (d << shift)
    mask = (gt | (eq & (lane <= q))).astype(jnp.bfloat16)

    m1 = jnp.roll(mask, 1, axis=1)
    for blk, s_ref in ((1024, s10_ref), (512, s9_ref), (256, s8_ref)):
        child = m1[:, blk:2 * blk]
        par = jnp.dot(child, s_ref[...],
                      preferred_element_type=jnp.float32)
        par = (par > 0.5).astype(jnp.bfloat16)
        par = jnp.pad(par, ((0, 0), (blk // 2, NP - blk)))
        m1 = jnp.maximum(m1, par)
    top = jnp.dot(m1[:, 0:256], anc_ref[...],
                  preferred_element_type=jnp.float32)
    top = (top > 0.5).astype(jnp.bfloat16)
    m1 = jnp.maximum(m1, jnp.pad(top, ((0, 0), (0, NP - 256))))
    maskf = jnp.roll(m1, -1, axis=1).astype(jnp.float32)

    x_new = jnp.sign(u) * jnp.maximum(jnp.abs(u) - theta_s, 0.0) * maskf
    out_ref[...] = jnp.where(valid, x_new, 0.0)


def _pad_cols(a, np_):
    return jnp.pad(a, ((0, 0), (0, np_ - a.shape[1])))


def kernel(y, A, W, A_pinv, c1, c2, c3, parent, depth):
    A_T = jnp.pad(A.T, ((0, NP - N), (0, 0)))
    Apinv_T = _pad_cols(A_pinv.T, NP)
    Wp = _pad_cols(W, NP)
    dw = _pad_cols((RHO ** depth.astype(jnp.float32))[None, :], NP)

    def _level_sel(d):
        lo, hi = (1 << d) - 1, (1 << (d + 1)) - 1
        loc_par = parent[lo:hi] - ((1 << (d - 1)) - 1)
        jj = jnp.arange(1 << (d - 1), dtype=jnp.int32)[None, :]
        return (loc_par[:, None] == jj).astype(jnp.bfloat16)

    s10, s9, s8 = _level_sel(10), _level_sel(9), _level_sel(8)
    cur = jnp.arange(255, dtype=jnp.int32)
    aa = jnp.arange(255, dtype=jnp.int32)[None, :]
    anc = jnp.zeros((255, 255), jnp.bool_)
    for _ in range(8):
        anc = anc | (cur[:, None] == aa)
        cur = parent[cur]
    anc255 = jnp.pad(anc.astype(jnp.bfloat16), ((1, 0), (1, 0)))

    c1a = jnp.abs(c1)
    c3a = jnp.abs(c3)
    sc = jnp.stack([c1a[0], jax.nn.sigmoid(c2[0]),
                    c3a[0], jnp.float32(0.0)]).reshape(1, 4)

    matmul_call = pl.pallas_call(
        _matmul_kernel,
        out_shape=(jax.ShapeDtypeStruct((B, NP), jnp.float32),
                   jax.ShapeDtypeStruct((B, NP), jnp.float32)),
    )
    select_call = pl.pallas_call(
        _select_kernel,
        out_shape=jax.ShapeDtypeStruct((B, NP), jnp.float32),
    )

    Apinv_y = (A_pinv @ y.T).T
    y_l1 = jnp.clip(jnp.sum(jnp.abs(Apinv_y), axis=-1, keepdims=True),
                    1e-12, None)

    x = jnp.zeros((B, NP), jnp.float32)
    x_prev = jnp.zeros_like(x)
    for k in range(NUM_LAYERS):
        Apinv_res, u = matmul_call(y, A_T, Apinv_T, Wp, sc, x, x_prev)
        Apr = Apinv_res[:, :N]
        res_l1 = jnp.clip(jnp.sum(jnp.abs(Apr), axis=-1, keepdims=True),
                          1e-12, None)
        residual_ratio = jnp.clip(res_l1 / y_l1, 0.0, 1.0)
        theta = c1a * residual_ratio
        layer_progress = float(k + 1) / float(NUM_LAYERS)
        ratio = jnp.clip(y_l1 / res_l1, 1.0, None)
        log_ratio = jnp.clip(jnp.log(ratio), 0.0, None)
        signal_estimate = jax.nn.sigmoid(log_ratio - 1.0)
        K_tree = jnp.clip(c3a * float(N) *
                          jnp.maximum(signal_estimate, layer_progress),
                          1.0, float(N) * 0.6)
        theta_s = jnp.mean(theta)
        Kv = jnp.maximum(jnp.floor(jnp.mean(K_tree)), 1.0)
        x_new = select_call(u, dw, s10, s9, s8, anc255,
                            theta_s.reshape(1, 1), Kv.reshape(1, 1))
        x_prev = x
        x = x_new
    return x[:, :N]


# speedup vs baseline: 1.6802x; 1.0976x over previous
import jax
import jax.numpy as jnp
from jax.experimental import pallas as pl

M, N, B = 512, 2047, 64
NP = 2048
NUM_LAYERS = 16
RHO = 0.5
MAX_DEPTH = 10


def _matmul_body(y, A_T, Apinv_T, W, sigc2, x, x_prev):
    residual = y - jnp.dot(x, A_T, preferred_element_type=jnp.float32)
    apr = jnp.dot(residual, Apinv_T, preferred_element_type=jnp.float32)
    active = jnp.sum((jnp.abs(x) > 1e-6).astype(jnp.float32),
                     axis=-1, keepdims=True)
    beta = sigc2 * (active / float(N))
    z = x + beta * (x - x_prev)
    residual2 = y - jnp.dot(z, A_T, preferred_element_type=jnp.float32)
    u = z + jnp.dot(residual2, W, preferred_element_type=jnp.float32)
    return apr, u


def _select_body(u, dw, s10, s9, s8, anc255, theta_s, Kv):
    lane = jax.lax.broadcasted_iota(jnp.int32, (1, NP), 1)
    valid = lane < N

    s = jnp.abs(u) * dw
    s_bits = jax.lax.bitcast_convert_type(s, jnp.int32)
    s_bits = jnp.where(valid, s_bits, -1)

    def cnt_ge(t):
        return jnp.sum((s_bits >= t).astype(jnp.int32),
                       axis=-1, keepdims=True)

    p = jnp.where(cnt_ge(jnp.full((B, 1), 1 << 30, jnp.int32)) >= Kv,
                  jnp.full((B, 1), 1 << 30, jnp.int32),
                  jnp.zeros((B, 1), jnp.int32))
    for shift in range(27, -1, -3):
        d = (cnt_ge(p + (1 << shift)) >= Kv).astype(jnp.int32)
        for k in range(2, 8):
            d = d + (cnt_ge(p + (k << shift)) >= Kv).astype(jnp.int32)
        p = p + (d << shift)
    v_bits = p
    gt = s_bits > v_bits
    eq = s_bits == v_bits
    c_gt = jnp.sum(gt.astype(jnp.int32), axis=-1, keepdims=True)
    need = Kv - c_gt

    def cnt_eq_le(t):
        return jnp.sum((eq & (lane <= t)).astype(jnp.int32),
                       axis=-1, keepdims=True)

    q = jnp.zeros((B, 1), jnp.int32)
    for shift in range(9, -1, -3):
        low1 = (1 << shift) - 1
        d = (cnt_eq_le(q + low1) < need).astype(jnp.int32)
        for k in range(1, 7):
            d = d + (cnt_eq_le(q + (k << shift) + low1) <
                     need).astype(jnp.int32)
        q = q + (d << shift)
    mask = (gt | (eq & (lane <= q))).astype(jnp.bfloat16)

    m1 = jnp.roll(mask, 1, axis=1)
    for blk, s_mat in ((1024, s10), (512, s9), (256, s8)):
        child = m1[:, blk:2 * blk]
        par = jnp.dot(child, s_mat, preferred_element_type=jnp.float32)
        par = (par > 0.5).astype(jnp.bfloat16)
        par = jnp.pad(par, ((0, 0), (blk // 2, NP - blk)))
        m1 = jnp.maximum(m1, par)
    top = jnp.dot(m1[:, 0:256], anc255, preferred_element_type=jnp.float32)
    top = (top > 0.5).astype(jnp.bfloat16)
    m1 = jnp.maximum(m1, jnp.pad(top, ((0, 0), (0, NP - 256))))
    maskf = jnp.roll(m1, -1, axis=1).astype(jnp.float32)

    x_new = jnp.sign(u) * jnp.maximum(jnp.abs(u) - theta_s, 0.0) * maskf
    return jnp.where(valid, x_new, 0.0)


def _matmul_kernel(y_ref, at_ref, apt_ref, w_ref, sc_ref, x_ref, xp_ref,
                   apr_ref, u_ref):
    apr, u = _matmul_body(y_ref[...], at_ref[...], apt_ref[...], w_ref[...],
                          sc_ref[0, 1], x_ref[...], xp_ref[...])
    apr_ref[...] = apr
    u_ref[...] = u


def _select_kernel(u_ref, dw_ref, s10_ref, s9_ref, s8_ref, anc_ref, ts_ref,
                   kv_ref, out_ref):
    out_ref[...] = _select_body(u_ref[...], dw_ref[...], s10_ref[...],
                                s9_ref[...], s8_ref[...], anc_ref[...],
                                ts_ref[0, 0], kv_ref[0, 0].astype(jnp.int32))


def _fused_layer_kernel(u_ref, dw_ref, s10_ref, s9_ref, s8_ref, anc_ref,
                        ts_ref, kv_ref, y_ref, at_ref, apt_ref, w_ref,
                        sc_ref, xp_ref, x_ref, apr_ref, u_out_ref):
    x = _select_body(u_ref[...], dw_ref[...], s10_ref[...], s9_ref[...],
                     s8_ref[...], anc_ref[...], ts_ref[0, 0],
                     kv_ref[0, 0].astype(jnp.int32))
    apr, u = _matmul_body(y_ref[...], at_ref[...], apt_ref[...], w_ref[...],
                          sc_ref[0, 1], x, xp_ref[...])
    x_ref[...] = x
    apr_ref[...] = apr
    u_out_ref[...] = u


def _pad_cols(a, np_):
    return jnp.pad(a, ((0, 0), (0, np_ - a.shape[1])))


def kernel(y, A, W, A_pinv, c1, c2, c3, parent, depth):
    A_T = jnp.pad(A.T, ((0, NP - N), (0, 0)))
    Apinv_T = _pad_cols(A_pinv.T, NP)
    Wp = _pad_cols(W, NP)
    dw = _pad_cols((RHO ** depth.astype(jnp.float32))[None, :], NP)

    def _level_sel(d):
        lo, hi = (1 << d) - 1, (1 << (d + 1)) - 1
        loc_par = parent[lo:hi] - ((1 << (d - 1)) - 1)
        jj = jnp.arange(1 << (d - 1), dtype=jnp.int32)[None, :]
        return (loc_par[:, None] == jj).astype(jnp.bfloat16)

    s10, s9, s8 = _level_sel(10), _level_sel(9), _level_sel(8)
    cur = jnp.arange(255, dtype=jnp.int32)
    aa = jnp.arange(255, dtype=jnp.int32)[None, :]
    anc = jnp.zeros((255, 255), jnp.bool_)
    for _ in range(8):
        anc = anc | (cur[:, None] == aa)
        cur = parent[cur]
    anc255 = jnp.pad(anc.astype(jnp.bfloat16), ((1, 0), (1, 0)))

    c1a = jnp.abs(c1)
    c3a = jnp.abs(c3)
    sc = jnp.stack([c1a[0], jax.nn.sigmoid(c2[0]),
                    c3a[0], jnp.float32(0.0)]).reshape(1, 4)

    fstruct = jax.ShapeDtypeStruct((B, NP), jnp.float32)
    matmul_call = pl.pallas_call(_matmul_kernel, out_shape=(fstruct, fstruct))
    select_call = pl.pallas_call(_select_kernel, out_shape=fstruct)
    fused_call = pl.pallas_call(_fused_layer_kernel,
                                out_shape=(fstruct, fstruct, fstruct))

    Apinv_y = (A_pinv @ y.T).T
    y_l1 = jnp.clip(jnp.sum(jnp.abs(Apinv_y), axis=-1, keepdims=True),
                    1e-12, None)

    def stats(apr, k):
        res_l1 = jnp.clip(jnp.sum(jnp.abs(apr[:, :N]), axis=-1,
                                  keepdims=True), 1e-12, None)
        residual_ratio = jnp.clip(res_l1 / y_l1, 0.0, 1.0)
        theta = c1a * residual_ratio
        layer_progress = float(k + 1) / float(NUM_LAYERS)
        ratio = jnp.clip(y_l1 / res_l1, 1.0, None)
        log_ratio = jnp.clip(jnp.log(ratio), 0.0, None)
        signal_estimate = jax.nn.sigmoid(log_ratio - 1.0)
        K_tree = jnp.clip(c3a * float(N) *
                          jnp.maximum(signal_estimate, layer_progress),
                          1.0, float(N) * 0.6)
        theta_s = jnp.mean(theta)
        Kv = jnp.maximum(jnp.floor(jnp.mean(K_tree)), 1.0)
        return theta_s.reshape(1, 1), Kv.reshape(1, 1)

    zero = jnp.zeros((B, NP), jnp.float32)
    apr, u = matmul_call(y, A_T, Apinv_T, Wp, sc, zero, zero)
    ts, kv = stats(apr, 0)
    x_prev = zero
    for k in range(1, NUM_LAYERS):
        x, apr, u = fused_call(u, dw, s10, s9, s8, anc255, ts, kv,
                               y, A_T, Apinv_T, Wp, sc, x_prev)
        ts, kv = stats(apr, k)
        x_prev = x
    x_final = select_call(u, dw, s10, s9, s8, anc255, ts, kv)
    return x_final[:, :N]
